# baseline (speedup 1.0000x reference)
import functools

import jax
import jax.numpy as jnp
from jax import lax
from jax.experimental import pallas as pl
from jax.experimental.pallas import tpu as pltpu
from jax.experimental.pallas import tpu_sc as plsc

N = 10000
NP = 10240
E = 320000
CHUNK = 128
EPAD = 2560 * CHUNK
NCH = EPAD // CHUNK
TCH = NCH // 32
TRASH = 10016
NG = 16
NSLOT = 4
LOOK = 2

_f32 = jnp.float32
_i32 = jnp.int32
_CP = pltpu.CompilerParams(needs_layout_passes=False,
                           use_tc_tiling_on_sc=False)


def _mesh():
    return plsc.VectorSubcoreMesh(core_axis_name="c", subcore_axis_name="s")


def _stream_pass(src_at, dst_acc, gidx, sidx, mbuf, gsems, ssems):
    def g_start(ch, sl):
        pltpu.async_copy(src_at(gidx.at[ch]), mbuf.at[sl], gsems[sl])

    def g_wait(ch, sl):
        pltpu.make_async_copy(src_at(gidx.at[ch]), mbuf.at[sl],
                              gsems[sl]).wait()

    def s_start(ch, sl):
        pltpu.async_copy(mbuf.at[sl], dst_acc.at[sidx.at[ch]], ssems[sl],
                         add=True)

    def s_wait(ch, sl):
        pltpu.make_async_copy(mbuf.at[sl], dst_acc.at[sidx.at[ch]],
                              ssems[sl]).wait()

    for i in range(LOOK):
        g_start(i, i)
    for i in range(NSLOT):
        if i >= LOOK:
            s_wait(i - LOOK, (i + LOOK) % NSLOT)
        g_start(i + LOOK, (i + LOOK) % NSLOT)
        g_wait(i, i)
        s_start(i, i)

    def _body(o, _):
        for i in range(NSLOT):
            ch = o * NSLOT + i
            s_wait(ch - LOOK, (i + LOOK) % NSLOT)
            g_start(ch + LOOK, (i + LOOK) % NSLOT)
            g_wait(ch, i)
            s_start(ch, i)
        return 0
    lax.fori_loop(1, TCH // NSLOT - 1, _body, 0)
    for i in range(NSLOT):
        ch = TCH - NSLOT + i
        s_wait(ch - LOOK, (i + LOOK) % NSLOT)
        if i < LOOK:
            g_start(ch + LOOK, (i + LOOK) % NSLOT)
        g_wait(ch, i)
        s_start(ch, i)
    for i in range(LOOK):
        ch = TCH - LOOK + i
        s_wait(ch, ch % NSLOT)


def _zero_slice(zbuf, acc, base, rows, zrows):
    for off in range(0, rows, zrows):
        pltpu.sync_copy(zbuf, acc.at[pl.ds(base + off, zrows)])


def _deg_body(rows_hbm, dego, rows_t, deg_p, wbuf, zbuf, ident, deg_sh):
    c = lax.axis_index("c")
    s = lax.axis_index("s")
    w = c * 16 + s
    iota = lax.iota(_i32, 16)
    ones = jnp.ones((16,), _f32)
    zeros16 = jnp.zeros((16,), _f32)

    pltpu.sync_copy(rows_hbm.at[pl.ds(w * TCH, TCH)], rows_t)

    def _init(o, _):
        zbuf[o, :] = zeros16
        deg_p[pl.ds(o * 16, 16)] = zeros16
        return 0
    lax.fori_loop(0, 640, _init, 0)
    for cc in range(8):
        for j in range(5):
            ident[cc, pl.ds(j * 16, 16)] = iota + (cc * 80 + j * 16)

    pltpu.sync_copy(zbuf.at[pl.ds(0, 40)], deg_sh.at[pl.ds(s * 40, 40)])
    plsc.subcore_barrier()

    def _hist(ch, _):
        for j in range(8):
            rv = rows_t[ch, pl.ds(j * 16, 16)]
            plsc.addupdate_scatter(deg_p, [rv], ones)
        return 0
    lax.fori_loop(0, TCH, _hist, 0)

    def _stage(r, _):
        wbuf[r, :] = deg_p[pl.ds(r * 16, 16)]
        return 0
    lax.fori_loop(0, 640, _stage, 0)
    for cc in range(8):
        pltpu.sync_copy(wbuf.at[pl.ds(cc * 80, 80)],
                        deg_sh.at[ident.at[cc]], add=True)
    plsc.subcore_barrier()
    pltpu.sync_copy(deg_sh.at[pl.ds(s * 40, 40)],
                    dego.at[c].at[pl.ds(s * 40, 40)])


@jax.jit
def _sc_deg(rows2d):
    return pl.kernel(
        _deg_body,
        jax.ShapeDtypeStruct((2, 640, 16), _f32),
        mesh=_mesh(),
        scratch_types=[
            pltpu.VMEM((TCH, CHUNK), _i32),
            pltpu.VMEM((NP,), _f32),
            pltpu.VMEM((640, 16), _f32),
            pltpu.VMEM((640, 16), _f32),
            pltpu.VMEM((8, 80), _i32),
            pltpu.VMEM_SHARED((640, 16), _f32),
        ],
        compiler_params=_CP, name="gcn_sc_deg")(rows2d)


def _u1acc_body(rows_hbm, cols_hbm, v0_hbm, v1_hbm, accu_o, acc1_o,
                rows_t, cols_t, zbuf, zbuf16, mbuf16, mbuf64,
                acc16, acc64, *sems):
    c = lax.axis_index("c")
    s = lax.axis_index("s")
    w = c * 16 + s
    gsems = sems[:NSLOT]
    ssems = sems[NSLOT:2 * NSLOT]
    zeros64 = jnp.zeros((16,), _f32)

    pltpu.sync_copy(rows_hbm.at[pl.ds(w * TCH, TCH)], rows_t)
    pltpu.sync_copy(cols_hbm.at[pl.ds(w * TCH, TCH)], cols_t)

    def _init(o, _):
        for j in range(4):
            zbuf[o, pl.ds(j * 16, 16)] = zeros64
        zbuf16[o, :] = zeros64
        return 0
    lax.fori_loop(0, 64, _init, 0)

    _zero_slice(zbuf16, acc16, s * 640, 640, 64)
    _zero_slice(zbuf, acc64, s * 640, 640, 64)
    plsc.subcore_barrier()

    _stream_pass(lambda idx: v0_hbm.at[idx], acc16, cols_t, rows_t,
                 mbuf16, gsems, ssems)
    _stream_pass(lambda idx: v1_hbm.at[idx], acc64, rows_t, cols_t,
                 mbuf64, gsems, ssems)
    plsc.subcore_barrier()

    pltpu.sync_copy(acc16.at[pl.ds(s * 640, 640)],
                    accu_o.at[c].at[pl.ds(s * 640, 640)])
    pltpu.sync_copy(acc64.at[pl.ds(s * 640, 640)],
                    acc1_o.at[c].at[pl.ds(s * 640, 640)])


@jax.jit
def _sc_u1acc1(rows2d, cols2d, v0, v1):
    return pl.kernel(
        _u1acc_body,
        (jax.ShapeDtypeStruct((2, NP, 16), _f32),
         jax.ShapeDtypeStruct((2, NP, 64), _f32)),
        mesh=_mesh(),
        scratch_types=[
            pltpu.VMEM((TCH, CHUNK), _i32),
            pltpu.VMEM((TCH, CHUNK), _i32),
            pltpu.VMEM((64, 64), _f32),
            pltpu.VMEM((64, 16), _f32),
            pltpu.VMEM((NSLOT, CHUNK, 16), _f32),
            pltpu.VMEM((NSLOT, CHUNK, 64), _f32),
            pltpu.VMEM_SHARED((NP, 16), _f32),
            pltpu.VMEM_SHARED((NP, 64), _f32),
        ] + [pltpu.SemaphoreType.DMA] * (2 * NSLOT),
        compiler_params=_CP, name="gcn_sc_u1acc1")(rows2d, cols2d, v0, v1)


def _pass64_body(rows_hbm, cols_hbm, v_hbm, acc_o,
                 rows_t, cols_t, zbuf, mbuf64, acc64, *sems):
    c = lax.axis_index("c")
    s = lax.axis_index("s")
    w = c * 16 + s
    gsems = sems[:NSLOT]
    ssems = sems[NSLOT:2 * NSLOT]
    zeros64 = jnp.zeros((16,), _f32)

    pltpu.sync_copy(rows_hbm.at[pl.ds(w * TCH, TCH)], rows_t)
    pltpu.sync_copy(cols_hbm.at[pl.ds(w * TCH, TCH)], cols_t)

    def _init(o, _):
        for j in range(4):
            zbuf[o, pl.ds(j * 16, 16)] = zeros64
        return 0
    lax.fori_loop(0, 64, _init, 0)
    _zero_slice(zbuf, acc64, s * 640, 640, 64)
    plsc.subcore_barrier()

    _stream_pass(lambda idx: v_hbm.at[idx], acc64, rows_t, cols_t,
                 mbuf64, gsems, ssems)
    plsc.subcore_barrier()

    pltpu.sync_copy(acc64.at[pl.ds(s * 640, 640)],
                    acc_o.at[c].at[pl.ds(s * 640, 640)])


@jax.jit
def _sc_pass64(rows2d, cols2d, v):
    return pl.kernel(
        _pass64_body,
        jax.ShapeDtypeStruct((2, NP, 64), _f32),
        mesh=_mesh(),
        scratch_types=[
            pltpu.VMEM((TCH, CHUNK), _i32),
            pltpu.VMEM((TCH, CHUNK), _i32),
            pltpu.VMEM((64, 64), _f32),
            pltpu.VMEM((NSLOT, CHUNK, 64), _f32),
            pltpu.VMEM_SHARED((NP, 64), _f32),
        ] + [pltpu.SemaphoreType.DMA] * (2 * NSLOT),
        compiler_params=_CP, name="gcn_sc_pass64")(rows2d, cols2d, v)


def _bmm(a, b):
    ab = a.astype(jnp.bfloat16).astype(_f32)
    bb = b.astype(jnp.bfloat16).astype(_f32)
    return lax.dot_general(ab, bb, (((1,), (1,)), ((), ())),
                           precision=lax.Precision.HIGHEST,
                           preferred_element_type=_f32)


def _tc1a_body(deg0_ref, deg1_ref, dinv_ref):
    deg = deg0_ref[...] + deg1_ref[...]
    dinv_ref[...] = jnp.where(deg > 0.0, lax.rsqrt(deg), 0.0)


@jax.jit
def _tc1a(deg0_pk, deg1_pk):
    return pl.pallas_call(
        _tc1a_body,
        out_shape=jax.ShapeDtypeStruct((640, 16), _f32),
    )(deg0_pk, deg1_pk)


def _tc1b_body(dinv_ref, batch_ref, x_ref, W1_ref, b1_ref,
               v0_ref, v1_ref):
    dinv = dinv_ref[...]
    iota = lax.broadcasted_iota(_i32, (1, NG), 1)
    v0_ref[...] = jnp.where(batch_ref[...] == iota, dinv, 0.0)
    z = _bmm(x_ref[...], W1_ref[...]) + b1_ref[...]
    v1_ref[pl.ds(0, N), :] = dinv[:N] * z
    v1_ref[pl.ds(N, NP - N), :] = jnp.zeros((NP - N, 64), _f32)


@jax.jit
def _tc1b(dinv_col, batch_col, x, W1, b1):
    return pl.pallas_call(
        _tc1b_body,
        out_shape=(jax.ShapeDtypeStruct((NP, NG), _f32),
                   jax.ShapeDtypeStruct((NP, 64), _f32)),
    )(dinv_col, batch_col, x, W1, b1)


def _tcmid_body(a0_ref, a1_ref, dinv_ref, W_ref, b_ref, v_ref):
    dinv = dinv_ref[...]
    h = dinv * (a0_ref[...] + a1_ref[...])
    z = _bmm(h, W_ref[...]) + b_ref[...]
    v_ref[...] = dinv * z


@jax.jit
def _tc_mid(a0, a1, dinv_col, W, b):
    return pl.pallas_call(
        _tcmid_body,
        out_shape=jax.ShapeDtypeStruct((NP, 64), _f32),
    )(a0, a1, dinv_col, W, b)


def _tcfin_body(au0_ref, au1_ref, a0_ref, a1_ref, dinv_ref, batch_ref,
                W4_ref, b4_ref, Wl_ref, bl_ref, out_ref):
    dinv = dinv_ref[...]
    U1 = dinv * (au0_ref[...] + au1_ref[...])
    h3 = dinv * (a0_ref[...] + a1_ref[...])
    z4 = _bmm(h3, W4_ref[...]) + b4_ref[...]
    pool = lax.dot_general(U1, z4, (((0,), (0,)), ((), ())),
                           precision=lax.Precision.HIGHEST,
                           preferred_element_type=_f32)
    b2d = batch_ref[...]
    n_max = jnp.float32(0.0)
    for gg in range(NG):
        n_max = jnp.maximum(n_max, jnp.sum((b2d == gg).astype(_f32)))
    x_new = pool / n_max
    xb = x_new.astype(jnp.bfloat16).astype(_f32)
    wb = Wl_ref[...].astype(jnp.bfloat16).astype(_f32)
    out_ref[...] = (jnp.sum(xb * wb, axis=1, keepdims=True)
                    + bl_ref[...])


@jax.jit
def _tc_final(au0, au1, a0, a1, dinv_col, batch_col, W4, b4, Wl, bl):
    return pl.pallas_call(
        _tcfin_body,
        out_shape=jax.ShapeDtypeStruct((NG, 1), _f32),
    )(au0, au1, a0, a1, dinv_col, batch_col, W4, b4, Wl, bl)


def kernel(x, edge_index, batch, W1, b1, W2, b2, W3, b3, W4, b4, Wl, bl):
    row = edge_index[0].astype(_i32)
    col = edge_index[1].astype(_i32)
    pad = jnp.full((EPAD - E,), TRASH, _i32)
    rows2d = jnp.concatenate([row, pad]).reshape(NCH, CHUNK)
    cols2d = jnp.concatenate([col, pad]).reshape(NCH, CHUNK)
    batch_col = jnp.concatenate(
        [batch.astype(_i32), jnp.full((NP - N,), -1, _i32)]).reshape(NP, 1)

    degp = _sc_deg(rows2d)
    dinv_pk = _tc1a(degp[0], degp[1])
    dinv_col = dinv_pk.reshape(NP, 1)
    v0, v1 = _tc1b(dinv_col, batch_col, x, W1, b1.reshape(1, 64))
    accu_p, acc1_p = _sc_u1acc1(rows2d, cols2d, v0, v1)
    v2 = _tc_mid(acc1_p[0], acc1_p[1], dinv_col, W2, b2.reshape(1, 64))
    acc2_p = _sc_pass64(rows2d, cols2d, v2)
    v3 = _tc_mid(acc2_p[0], acc2_p[1], dinv_col, W3, b3.reshape(1, 64))
    acc3_p = _sc_pass64(rows2d, cols2d, v3)
    out = _tc_final(accu_p[0], accu_p[1], acc3_p[0], acc3_p[1],
                    dinv_col, batch_col, W4, b4.reshape(1, 64),
                    Wl, bl.reshape(1, 1))
    return out


# speedup vs baseline: 2.3934x; 2.3934x over previous
import functools

import jax
import jax.numpy as jnp
from jax import lax
from jax.experimental import pallas as pl
from jax.experimental.pallas import tpu as pltpu
from jax.experimental.pallas import tpu_sc as plsc

N = 10000
NP = 10240
E = 320000
CHUNK = 128
EPAD = 2560 * CHUNK
NCH = EPAD // CHUNK
TCH = NCH // 32
TRASH = 10016
NG = 16
NSLOT = 4
LOOK = 2

_f32 = jnp.float32
_i32 = jnp.int32
_CP = pltpu.CompilerParams(needs_layout_passes=False,
                           use_tc_tiling_on_sc=False)


def _mesh():
    return plsc.VectorSubcoreMesh(core_axis_name="c", subcore_axis_name="s")


def _stream_pass(src_at, dst_acc, gidx, sidx, mbuf, gsems, ssems):
    def g_start(ch, sl):
        pltpu.async_copy(src_at(gidx.at[ch]), mbuf.at[sl], gsems[sl])

    def g_wait(ch, sl):
        pltpu.make_async_copy(src_at(gidx.at[ch]), mbuf.at[sl],
                              gsems[sl]).wait()

    def s_start(ch, sl):
        pltpu.async_copy(mbuf.at[sl], dst_acc.at[sidx.at[ch]], ssems[sl],
                         add=True)

    def s_wait(ch, sl):
        pltpu.make_async_copy(mbuf.at[sl], dst_acc.at[sidx.at[ch]],
                              ssems[sl]).wait()

    for i in range(LOOK):
        g_start(i, i)
    for i in range(NSLOT):
        if i >= LOOK:
            s_wait(i - LOOK, (i + LOOK) % NSLOT)
        g_start(i + LOOK, (i + LOOK) % NSLOT)
        g_wait(i, i)
        s_start(i, i)

    def _body(o, _):
        for i in range(NSLOT):
            ch = o * NSLOT + i
            s_wait(ch - LOOK, (i + LOOK) % NSLOT)
            g_start(ch + LOOK, (i + LOOK) % NSLOT)
            g_wait(ch, i)
            s_start(ch, i)
        return 0
    lax.fori_loop(1, TCH // NSLOT - 1, _body, 0)
    for i in range(NSLOT):
        ch = TCH - NSLOT + i
        s_wait(ch - LOOK, (i + LOOK) % NSLOT)
        if i < LOOK:
            g_start(ch + LOOK, (i + LOOK) % NSLOT)
        g_wait(ch, i)
        s_start(ch, i)
    for i in range(LOOK):
        ch = TCH - LOOK + i
        s_wait(ch, ch % NSLOT)


def _zero_slice(zbuf, acc, base, rows, zrows):
    for off in range(0, rows, zrows):
        pltpu.sync_copy(zbuf, acc.at[pl.ds(base + off, zrows)])


def _deg_body(rows_hbm, dego, rows_t, deg_p, wbuf, zbuf, ident, deg_sh):
    c = lax.axis_index("c")
    s = lax.axis_index("s")
    w = c * 16 + s
    iota = lax.iota(_i32, 16)
    ones = jnp.ones((16,), _f32)
    zeros16 = jnp.zeros((16,), _f32)

    pltpu.sync_copy(rows_hbm.at[pl.ds(w * TCH, TCH)], rows_t)

    def _init(o, _):
        zbuf[o, :] = zeros16
        deg_p[pl.ds(o * 16, 16)] = zeros16
        return 0
    lax.fori_loop(0, 640, _init, 0)
    for cc in range(8):
        for j in range(5):
            ident[cc, pl.ds(j * 16, 16)] = iota + (cc * 80 + j * 16)

    pltpu.sync_copy(zbuf.at[pl.ds(0, 40)], deg_sh.at[pl.ds(s * 40, 40)])
    plsc.subcore_barrier()

    def _hist(ch, _):
        for j in range(8):
            rv = rows_t[ch, pl.ds(j * 16, 16)]
            plsc.addupdate_scatter(deg_p, [rv], ones)
        return 0
    lax.fori_loop(0, TCH, _hist, 0)

    def _stage(r, _):
        wbuf[r, :] = deg_p[pl.ds(r * 16, 16)]
        return 0
    lax.fori_loop(0, 640, _stage, 0)
    for cc in range(8):
        pltpu.sync_copy(wbuf.at[pl.ds(cc * 80, 80)],
                        deg_sh.at[ident.at[cc]], add=True)
    plsc.subcore_barrier()
    pltpu.sync_copy(deg_sh.at[pl.ds(s * 40, 40)],
                    dego.at[c].at[pl.ds(s * 40, 40)])


@jax.jit
def _sc_deg(rows2d):
    return pl.kernel(
        _deg_body,
        jax.ShapeDtypeStruct((2, 640, 16), _f32),
        mesh=_mesh(),
        scratch_types=[
            pltpu.VMEM((TCH, CHUNK), _i32),
            pltpu.VMEM((NP,), _f32),
            pltpu.VMEM((640, 16), _f32),
            pltpu.VMEM((640, 16), _f32),
            pltpu.VMEM((8, 80), _i32),
            pltpu.VMEM_SHARED((640, 16), _f32),
        ],
        compiler_params=_CP, name="gcn_sc_deg")(rows2d)


def _u1acc_body(rows_hbm, cols_hbm, v0_hbm, v1_hbm, accu_o, acc1_o,
                rows_t, cols_t, zbuf, zbuf16, mbuf16, mbuf64,
                acc16, acc64, *sems):
    c = lax.axis_index("c")
    s = lax.axis_index("s")
    w = c * 16 + s
    gsems = sems[:NSLOT]
    ssems = sems[NSLOT:2 * NSLOT]
    zeros64 = jnp.zeros((16,), _f32)

    pltpu.sync_copy(rows_hbm.at[pl.ds(w * TCH, TCH)], rows_t)
    pltpu.sync_copy(cols_hbm.at[pl.ds(w * TCH, TCH)], cols_t)

    def _init(o, _):
        for j in range(4):
            zbuf[o, pl.ds(j * 16, 16)] = zeros64
        zbuf16[o, :] = zeros64
        return 0
    lax.fori_loop(0, 64, _init, 0)

    _zero_slice(zbuf16, acc16, s * 640, 640, 64)
    _zero_slice(zbuf, acc64, s * 640, 640, 64)
    plsc.subcore_barrier()

    _stream_pass(lambda idx: v0_hbm.at[idx], acc16, cols_t, rows_t,
                 mbuf16, gsems, ssems)
    _stream_pass(lambda idx: v1_hbm.at[idx], acc64, rows_t, cols_t,
                 mbuf64, gsems, ssems)
    plsc.subcore_barrier()

    pltpu.sync_copy(acc16.at[pl.ds(s * 640, 640)],
                    accu_o.at[c].at[pl.ds(s * 640, 640)])
    pltpu.sync_copy(acc64.at[pl.ds(s * 640, 640)],
                    acc1_o.at[c].at[pl.ds(s * 640, 640)])


@jax.jit
def _sc_u1acc1(rows2d, cols2d, v0, v1):
    return pl.kernel(
        _u1acc_body,
        (jax.ShapeDtypeStruct((2, NP, 16), _f32),
         jax.ShapeDtypeStruct((2, NP, 64), _f32)),
        mesh=_mesh(),
        scratch_types=[
            pltpu.VMEM((TCH, CHUNK), _i32),
            pltpu.VMEM((TCH, CHUNK), _i32),
            pltpu.VMEM((64, 64), _f32),
            pltpu.VMEM((64, 16), _f32),
            pltpu.VMEM((NSLOT, CHUNK, 16), _f32),
            pltpu.VMEM((NSLOT, CHUNK, 64), _f32),
            pltpu.VMEM_SHARED((NP, 16), _f32),
            pltpu.VMEM_SHARED((NP, 64), _f32),
        ] + [pltpu.SemaphoreType.DMA] * (2 * NSLOT),
        compiler_params=_CP, name="gcn_sc_u1acc1")(rows2d, cols2d, v0, v1)


def _pass64_body(rows_hbm, cols_hbm, v_hbm, acc_o,
                 rows_t, cols_t, zbuf, mbuf64, acc64, *sems):
    c = lax.axis_index("c")
    s = lax.axis_index("s")
    w = c * 16 + s
    gsems = sems[:NSLOT]
    ssems = sems[NSLOT:2 * NSLOT]
    zeros64 = jnp.zeros((16,), _f32)

    pltpu.sync_copy(rows_hbm.at[pl.ds(w * TCH, TCH)], rows_t)
    pltpu.sync_copy(cols_hbm.at[pl.ds(w * TCH, TCH)], cols_t)

    def _init(o, _):
        for j in range(4):
            zbuf[o, pl.ds(j * 16, 16)] = zeros64
        return 0
    lax.fori_loop(0, 64, _init, 0)
    _zero_slice(zbuf, acc64, s * 640, 640, 64)
    plsc.subcore_barrier()

    _stream_pass(lambda idx: v_hbm.at[idx], acc64, rows_t, cols_t,
                 mbuf64, gsems, ssems)
    plsc.subcore_barrier()

    pltpu.sync_copy(acc64.at[pl.ds(s * 640, 640)],
                    acc_o.at[c].at[pl.ds(s * 640, 640)])


@jax.jit
def _sc_pass64(rows2d, cols2d, v):
    return pl.kernel(
        _pass64_body,
        jax.ShapeDtypeStruct((2, NP, 64), _f32),
        mesh=_mesh(),
        scratch_types=[
            pltpu.VMEM((TCH, CHUNK), _i32),
            pltpu.VMEM((TCH, CHUNK), _i32),
            pltpu.VMEM((64, 64), _f32),
            pltpu.VMEM((NSLOT, CHUNK, 64), _f32),
            pltpu.VMEM_SHARED((NP, 64), _f32),
        ] + [pltpu.SemaphoreType.DMA] * (2 * NSLOT),
        compiler_params=_CP, name="gcn_sc_pass64")(rows2d, cols2d, v)


def _bmm(a, b):
    ab = a.astype(jnp.bfloat16).astype(_f32)
    bb = b.astype(jnp.bfloat16).astype(_f32)
    return lax.dot_general(ab, bb, (((1,), (1,)), ((), ())),
                           precision=lax.Precision.HIGHEST,
                           preferred_element_type=_f32)


def _tc1a_body(deg0_ref, deg1_ref, dinv_ref):
    deg = deg0_ref[...] + deg1_ref[...]
    dinv_ref[...] = jnp.where(deg > 0.0, lax.rsqrt(deg), 0.0)


@jax.jit
def _tc1a(deg0_pk, deg1_pk):
    return pl.pallas_call(
        _tc1a_body,
        out_shape=jax.ShapeDtypeStruct((640, 16), _f32),
    )(deg0_pk, deg1_pk)


def _tc1b_body(dinv_ref, batch_ref, x_ref, W1_ref, b1_ref,
               v0_ref, v1_ref):
    dinv = dinv_ref[...]
    iota = lax.broadcasted_iota(_i32, (1, NG), 1)
    v0_ref[...] = jnp.where(batch_ref[...] == iota, dinv, 0.0)
    z = _bmm(x_ref[...], W1_ref[...]) + b1_ref[...]
    v1_ref[pl.ds(0, N), :] = dinv[:N] * z
    v1_ref[pl.ds(N, NP - N), :] = jnp.zeros((NP - N, 64), _f32)


@jax.jit
def _tc1b(dinv_col, batch_col, x, W1, b1):
    return pl.pallas_call(
        _tc1b_body,
        out_shape=(jax.ShapeDtypeStruct((NP, NG), _f32),
                   jax.ShapeDtypeStruct((NP, 64), _f32)),
    )(dinv_col, batch_col, x, W1, b1)


def _tcmid_body(a0_ref, a1_ref, dinv_ref, W_ref, b_ref, v_ref):
    dinv = dinv_ref[...]
    h = dinv * (a0_ref[...] + a1_ref[...])
    z = _bmm(h, W_ref[...]) + b_ref[...]
    v_ref[...] = dinv * z


@jax.jit
def _tc_mid(a0, a1, dinv_col, W, b):
    return pl.pallas_call(
        _tcmid_body,
        out_shape=jax.ShapeDtypeStruct((NP, 64), _f32),
    )(a0, a1, dinv_col, W, b)


def _tcfin_body(au0_ref, au1_ref, a0_ref, a1_ref, dinv_ref, batch_ref,
                W4_ref, b4_ref, Wl_ref, bl_ref, out_ref):
    dinv = dinv_ref[...]
    U1 = dinv * (au0_ref[...] + au1_ref[...])
    h3 = dinv * (a0_ref[...] + a1_ref[...])
    z4 = _bmm(h3, W4_ref[...]) + b4_ref[...]
    pool = lax.dot_general(U1, z4, (((0,), (0,)), ((), ())),
                           precision=lax.Precision.HIGHEST,
                           preferred_element_type=_f32)
    b2d = batch_ref[...]
    n_max = jnp.float32(0.0)
    for gg in range(NG):
        n_max = jnp.maximum(n_max, jnp.sum((b2d == gg).astype(_f32)))
    x_new = pool / n_max
    xb = x_new.astype(jnp.bfloat16).astype(_f32)
    wb = Wl_ref[...].astype(jnp.bfloat16).astype(_f32)
    out_ref[...] = (jnp.sum(xb * wb, axis=1, keepdims=True)
                    + bl_ref[...])


@jax.jit
def _tc_final(au0, au1, a0, a1, dinv_col, batch_col, W4, b4, Wl, bl):
    return pl.pallas_call(
        _tcfin_body,
        out_shape=jax.ShapeDtypeStruct((NG, 1), _f32),
    )(au0, au1, a0, a1, dinv_col, batch_col, W4, b4, Wl, bl)


def kernel(x, edge_index, batch, W1, b1, W2, b2, W3, b3, W4, b4, Wl, bl):
    row = edge_index[0].astype(_i32)
    col = edge_index[1].astype(_i32)
    pad = N + (jnp.arange(EPAD - E, dtype=_i32) % (NP - N))
    rows2d = jnp.concatenate([row, pad]).reshape(NCH, CHUNK)
    cols2d = jnp.concatenate([col, pad]).reshape(NCH, CHUNK)
    batch_col = jnp.concatenate(
        [batch.astype(_i32), jnp.full((NP - N,), -1, _i32)]).reshape(NP, 1)

    degp = _sc_deg(rows2d)
    dinv_pk = _tc1a(degp[0], degp[1])
    dinv_col = dinv_pk.reshape(NP, 1)
    v0, v1 = _tc1b(dinv_col, batch_col, x, W1, b1.reshape(1, 64))
    accu_p, acc1_p = _sc_u1acc1(rows2d, cols2d, v0, v1)
    v2 = _tc_mid(acc1_p[0], acc1_p[1], dinv_col, W2, b2.reshape(1, 64))
    acc2_p = _sc_pass64(rows2d, cols2d, v2)
    v3 = _tc_mid(acc2_p[0], acc2_p[1], dinv_col, W3, b3.reshape(1, 64))
    acc3_p = _sc_pass64(rows2d, cols2d, v3)
    out = _tc_final(accu_p[0], accu_p[1], acc3_p[0], acc3_p[1],
                    dinv_col, batch_col, W4, b4.reshape(1, 64),
                    Wl, bl.reshape(1, 1))
    return out
